# baseline (speedup 1.0000x reference)
import jax
import jax.numpy as jnp
from jax.experimental import pallas as pl
from jax.experimental.pallas import tpu as pltpu

ORIG_LEN = 4096
KEPT_LEN = 3584
IMG_START = 64
IMG_LEN = 3072
D_MODEL = 4096
HEAD_DIM = 128
H_STRIDE = 16
FRAME_STRIDE = 384

IDX_BLK = 512
STEN_BLK = 256
HALO = 16


def _idx_body(keep_ref, rid_ref, g_ref, m_ref):
    keep = keep_ref[...]
    rid = rid_ref[...]
    lt = (keep < rid).astype(jnp.int32)
    eq = (keep == rid).astype(jnp.int32)
    rank = jnp.sum(lt, axis=1, keepdims=True)
    member = jnp.max(eq, axis=1, keepdims=True)
    g_ref[...] = jnp.minimum(rank, KEPT_LEN - 1)
    m_ref[...] = member


def _idx_call(keep2, ridall):
    n = ridall.shape[0]
    return pl.pallas_call(
        _idx_body,
        grid=(n // IDX_BLK,),
        in_specs=[
            pl.BlockSpec((1, KEPT_LEN), lambda i: (0, 0)),
            pl.BlockSpec((IDX_BLK, 1), lambda i: (i, 0)),
        ],
        out_specs=[
            pl.BlockSpec((IDX_BLK, 1), lambda i: (i, 0)),
            pl.BlockSpec((IDX_BLK, 1), lambda i: (i, 0)),
        ],
        out_shape=[
            jax.ShapeDtypeStruct((n, 1), jnp.int32),
            jax.ShapeDtypeStruct((n, 1), jnp.int32),
        ],
    )(keep2, ridall)


def _gather_body(g_ref, m_ref, s_ref, ms_ref, h_ref, c_ref, si_ref,
                 o_ref, oc_ref, os_ref):
    i = pl.program_id(0)
    mf = m_ref[i].astype(jnp.float32)
    msf = ms_ref[i].astype(jnp.float32)
    o_ref[...] = h_ref[...] * mf
    oc_ref[...] = c_ref[...] * msf
    os_ref[...] = si_ref[...] * msf


def _gather_call(g, m, s, ms, hs3, cos3, sin3):
    grid_spec = pltpu.PrefetchScalarGridSpec(
        num_scalar_prefetch=4,
        grid=(ORIG_LEN,),
        in_specs=[
            pl.BlockSpec((1, 1, D_MODEL), lambda i, g, m, s, ms: (g[i], 0, 0)),
            pl.BlockSpec((1, 1, HEAD_DIM), lambda i, g, m, s, ms: (s[i], 0, 0)),
            pl.BlockSpec((1, 1, HEAD_DIM), lambda i, g, m, s, ms: (s[i], 0, 0)),
        ],
        out_specs=[
            pl.BlockSpec((1, 1, D_MODEL), lambda i, g, m, s, ms: (i, 0, 0)),
            pl.BlockSpec((1, 1, HEAD_DIM), lambda i, g, m, s, ms: (i, 0, 0)),
            pl.BlockSpec((1, 1, HEAD_DIM), lambda i, g, m, s, ms: (i, 0, 0)),
        ],
    )
    return pl.pallas_call(
        _gather_body,
        grid_spec=grid_spec,
        out_shape=[
            jax.ShapeDtypeStruct((ORIG_LEN, 1, D_MODEL), jnp.float32),
            jax.ShapeDtypeStruct((ORIG_LEN, 1, HEAD_DIM), jnp.float32),
            jax.ShapeDtypeStruct((ORIG_LEN, 1, HEAD_DIM), jnp.float32),
        ],
    )(g, m, s, ms, hs3, cos3, sin3)


def _stencil_body(pre_ref, main_ref, post_ref, out_ref):
    pid = pl.program_id(0)
    W = jnp.concatenate([pre_ref[...], main_ref[...], post_ref[...]], axis=0)
    S = jnp.sign(W)
    T = jnp.concatenate(
        [jnp.zeros((HALO, 1), jnp.float32),
         jnp.sum(S[HALO:] * S[:-HALO], axis=1, keepdims=True)], axis=0)
    L = jnp.concatenate(
        [jnp.zeros((1, 1), jnp.float32),
         jnp.sum(S[1:] * S[:-1], axis=1, keepdims=True)], axis=0)
    w = jax.lax.broadcasted_iota(jnp.int32, (STEN_BLK + 2 * HALO, 1), 0)
    r = pid * STEN_BLK - HALO + w
    j = r - IMG_START
    notmargin = ((j >= FRAME_STRIDE) & (j % FRAME_STRIDE >= H_STRIDE)
                 & (j % H_STRIDE != 0) & (j < IMG_LEN))
    fany = notmargin & (jnp.maximum(T, L) >= 0.0)
    left_wins = L > T
    f1 = (fany & left_wins).astype(jnp.float32)
    f16 = (fany & jnp.logical_not(left_wins)).astype(jnp.float32)
    Wm = W[HALO:HALO + STEN_BLK]
    f1p = f1[HALO + 1:HALO + 1 + STEN_BLK]
    f16p = f16[2 * HALO:2 * HALO + STEN_BLK]
    num = (Wm + f1p * W[HALO + 1:HALO + 1 + STEN_BLK]
           + f16p * W[2 * HALO:2 * HALO + STEN_BLK])
    den = 1.0 + f1p + f16p
    rm = r[HALO:HALO + STEN_BLK]
    is_img = (rm >= IMG_START) & (rm < IMG_START + IMG_LEN)
    fm = fany[HALO:HALO + STEN_BLK]
    merged = jnp.where(fm, 0.0, num / den)
    out_ref[...] = jnp.where(is_img, merged, Wm)


def _stencil_call(out1):
    nblk = ORIG_LEN // STEN_BLK
    nhalo = ORIG_LEN // HALO
    return pl.pallas_call(
        _stencil_body,
        grid=(nblk,),
        in_specs=[
            pl.BlockSpec((HALO, D_MODEL),
                         lambda i: (jnp.maximum(i * (STEN_BLK // HALO) - 1, 0), 0)),
            pl.BlockSpec((STEN_BLK, D_MODEL), lambda i: (i, 0)),
            pl.BlockSpec((HALO, D_MODEL),
                         lambda i: (jnp.minimum(i * (STEN_BLK // HALO) + (STEN_BLK // HALO),
                                                nhalo - 1), 0)),
        ],
        out_specs=pl.BlockSpec((STEN_BLK, D_MODEL), lambda i: (i, 0)),
        out_shape=jax.ShapeDtypeStruct((ORIG_LEN, D_MODEL), jnp.float32),
    )(out1, out1, out1)


def _zeros_body(o_ref):
    o_ref[...] = jnp.zeros_like(o_ref)


def _zeros_call():
    return pl.pallas_call(
        _zeros_body,
        grid=(8,),
        out_specs=pl.BlockSpec((ORIG_LEN // 8, ORIG_LEN), lambda i: (i, 0)),
        out_shape=jax.ShapeDtypeStruct((ORIG_LEN, ORIG_LEN), jnp.float32),
    )()


def kernel(hidden_states, pos_cos, pos_sin, attention_mask, keep_ids, merged_ids):
    hs3 = hidden_states.reshape(KEPT_LEN, 1, D_MODEL)
    cos3 = pos_cos.reshape(KEPT_LEN, 1, HEAD_DIM)
    sin3 = pos_sin.reshape(KEPT_LEN, 1, HEAD_DIM)
    keep2 = keep_ids.astype(jnp.int32).reshape(1, KEPT_LEN)
    r_out = jnp.arange(ORIG_LEN, dtype=jnp.int32)
    rid = jnp.concatenate([
        r_out[:IMG_START],
        IMG_START + merged_ids.astype(jnp.int32),
        r_out[IMG_START + IMG_LEN:],
    ])
    ridall = jnp.concatenate([rid, r_out]).reshape(2 * ORIG_LEN, 1)
    g2, m2 = _idx_call(keep2, ridall)
    g = g2[:ORIG_LEN, 0]
    m = m2[:ORIG_LEN, 0]
    s = g2[ORIG_LEN:, 0]
    ms = m2[ORIG_LEN:, 0]
    out1, cosf, sinf = _gather_call(g, m, s, ms, hs3, cos3, sin3)
    hid = _stencil_call(out1.reshape(ORIG_LEN, D_MODEL))
    attn = _zeros_call()
    return (hid.reshape(1, ORIG_LEN, D_MODEL),
            cosf.reshape(1, ORIG_LEN, HEAD_DIM),
            sinf.reshape(1, ORIG_LEN, HEAD_DIM),
            attn.reshape(1, 1, ORIG_LEN, ORIG_LEN))


# speedup vs baseline: 9.2819x; 9.2819x over previous
import functools

import jax
import jax.numpy as jnp
from jax import lax
from jax.experimental import pallas as pl
from jax.experimental.pallas import tpu as pltpu
from jax.experimental.pallas import tpu_sc as plsc

ORIG_LEN = 4096
KEPT_LEN = 3584
IMG_START = 64
IMG_LEN = 3072
D_MODEL = 4096
HEAD_DIM = 128
H_STRIDE = 16
FRAME_STRIDE = 384

IDX_BLK = 512
STEN_BLK = 256
HALO = 16

NC = 2
NS = 16
NW = NC * NS
ROWS_PER_W = ORIG_LEN // NW
CHUNK = 8
NCHUNK = ROWS_PER_W // CHUNK


def _idx_body(keep_ref, rid_ref, g_ref, m_ref):
    keep = keep_ref[...]
    rid = rid_ref[...]
    lt = (keep < rid).astype(jnp.int32)
    eq = (keep == rid).astype(jnp.int32)
    rank = jnp.sum(lt, axis=1, keepdims=True)
    member = jnp.max(eq, axis=1, keepdims=True)
    g_ref[...] = jnp.minimum(rank, KEPT_LEN - 1)
    m_ref[...] = member


def _idx_call(keep2, ridall):
    n = ridall.shape[0]
    return pl.pallas_call(
        _idx_body,
        grid=(n // IDX_BLK,),
        in_specs=[
            pl.BlockSpec((1, KEPT_LEN), lambda i: (0, 0)),
            pl.BlockSpec((IDX_BLK, 1), lambda i: (i, 0)),
        ],
        out_specs=[
            pl.BlockSpec((IDX_BLK, 1), lambda i: (i, 0)),
            pl.BlockSpec((IDX_BLK, 1), lambda i: (i, 0)),
        ],
        out_shape=[
            jax.ShapeDtypeStruct((n, 1), jnp.int32),
            jax.ShapeDtypeStruct((n, 1), jnp.int32),
        ],
    )(keep2, ridall)


def _make_sc_gather():
    mesh = plsc.VectorSubcoreMesh(core_axis_name="c", subcore_axis_name="s")

    @functools.partial(
        pl.kernel,
        mesh=mesh,
        out_type=[
            jax.ShapeDtypeStruct((ORIG_LEN, D_MODEL), jnp.float32),
            jax.ShapeDtypeStruct((ORIG_LEN, HEAD_DIM), jnp.float32),
            jax.ShapeDtypeStruct((ORIG_LEN, HEAD_DIM), jnp.float32),
        ],
        scratch_types=[
            pltpu.VMEM((CHUNK,), jnp.int32),
            pltpu.VMEM((CHUNK, D_MODEL), jnp.float32),
            pltpu.VMEM((ROWS_PER_W,), jnp.int32),
            pltpu.VMEM((ROWS_PER_W, HEAD_DIM), jnp.float32),
            pltpu.SemaphoreType.DMA,
        ],
    )
    def sc_gather(g_hbm, s_hbm, hs_hbm, cos_hbm, sin_hbm,
                  out_hbm, cosr_hbm, sinr_hbm,
                  idx_v, rows_v, idx2_v, crow_v, sem):
        wid = lax.axis_index("s") * NC + lax.axis_index("c")
        base = wid * ROWS_PER_W
        for chunk in range(NCHUNK):
            r0 = base + chunk * CHUNK
            pltpu.sync_copy(g_hbm.at[pl.ds(r0, CHUNK)], idx_v)
            pltpu.async_copy(hs_hbm.at[idx_v], rows_v, sem).wait()
            pltpu.sync_copy(rows_v, out_hbm.at[pl.ds(r0, CHUNK)])
        pltpu.sync_copy(s_hbm.at[pl.ds(base, ROWS_PER_W)], idx2_v)
        pltpu.async_copy(cos_hbm.at[idx2_v], crow_v, sem).wait()
        pltpu.sync_copy(crow_v, cosr_hbm.at[pl.ds(base, ROWS_PER_W)])
        pltpu.async_copy(sin_hbm.at[idx2_v], crow_v, sem).wait()
        pltpu.sync_copy(crow_v, sinr_hbm.at[pl.ds(base, ROWS_PER_W)])

    return sc_gather


_sc_gather = _make_sc_gather()


def _stencil_body(pre_ref, main_ref, post_ref, mpre_ref, mmain_ref, mpost_ref,
                  out_ref):
    pid = pl.program_id(0)
    Mw = jnp.concatenate([mpre_ref[...], mmain_ref[...], mpost_ref[...]],
                         axis=0)
    W = jnp.concatenate([pre_ref[...], main_ref[...], post_ref[...]], axis=0)
    W = W * Mw
    S = jnp.sign(W)
    T = jnp.concatenate(
        [jnp.zeros((HALO, 1), jnp.float32),
         jnp.sum(S[HALO:] * S[:-HALO], axis=1, keepdims=True)], axis=0)
    L = jnp.concatenate(
        [jnp.zeros((1, 1), jnp.float32),
         jnp.sum(S[1:] * S[:-1], axis=1, keepdims=True)], axis=0)
    w = jax.lax.broadcasted_iota(jnp.int32, (STEN_BLK + 2 * HALO, 1), 0)
    r = pid * STEN_BLK - HALO + w
    j = r - IMG_START
    notmargin = ((j >= FRAME_STRIDE) & (j % FRAME_STRIDE >= H_STRIDE)
                 & (j % H_STRIDE != 0) & (j < IMG_LEN))
    fany = notmargin & (jnp.maximum(T, L) >= 0.0)
    left_wins = L > T
    f1 = (fany & left_wins).astype(jnp.float32)
    f16 = (fany & jnp.logical_not(left_wins)).astype(jnp.float32)
    Wm = W[HALO:HALO + STEN_BLK]
    f1p = f1[HALO + 1:HALO + 1 + STEN_BLK]
    f16p = f16[2 * HALO:2 * HALO + STEN_BLK]
    num = (Wm + f1p * W[HALO + 1:HALO + 1 + STEN_BLK]
           + f16p * W[2 * HALO:2 * HALO + STEN_BLK])
    den = 1.0 + f1p + f16p
    rm = r[HALO:HALO + STEN_BLK]
    is_img = (rm >= IMG_START) & (rm < IMG_START + IMG_LEN)
    fm = fany[HALO:HALO + STEN_BLK]
    merged = jnp.where(fm, 0.0, num / den)
    out_ref[...] = jnp.where(is_img, merged, Wm)


def _stencil_call(out1, msk):
    nblk = ORIG_LEN // STEN_BLK
    nhalo = ORIG_LEN // HALO
    hpre = lambda i: (jnp.maximum(i * (STEN_BLK // HALO) - 1, 0), 0)
    hpost = lambda i: (jnp.minimum(i * (STEN_BLK // HALO) + (STEN_BLK // HALO),
                                   nhalo - 1), 0)
    return pl.pallas_call(
        _stencil_body,
        grid=(nblk,),
        in_specs=[
            pl.BlockSpec((HALO, D_MODEL), hpre),
            pl.BlockSpec((STEN_BLK, D_MODEL), lambda i: (i, 0)),
            pl.BlockSpec((HALO, D_MODEL), hpost),
            pl.BlockSpec((HALO, 1), hpre),
            pl.BlockSpec((STEN_BLK, 1), lambda i: (i, 0)),
            pl.BlockSpec((HALO, 1), hpost),
        ],
        out_specs=pl.BlockSpec((STEN_BLK, D_MODEL), lambda i: (i, 0)),
        out_shape=jax.ShapeDtypeStruct((ORIG_LEN, D_MODEL), jnp.float32),
    )(out1, out1, out1, msk, msk, msk)


def _post_body(c_ref, s_ref, m_ref, attn_ref, oc_ref, os_ref):
    mf = m_ref[...]
    attn_ref[...] = jnp.zeros_like(attn_ref)
    oc_ref[...] = c_ref[...] * mf
    os_ref[...] = s_ref[...] * mf


def _post_call(cosr, sinr, msf):
    nblk = 8
    blk = ORIG_LEN // nblk
    return pl.pallas_call(
        _post_body,
        grid=(nblk,),
        in_specs=[
            pl.BlockSpec((blk, HEAD_DIM), lambda i: (i, 0)),
            pl.BlockSpec((blk, HEAD_DIM), lambda i: (i, 0)),
            pl.BlockSpec((blk, 1), lambda i: (i, 0)),
        ],
        out_specs=[
            pl.BlockSpec((blk, ORIG_LEN), lambda i: (i, 0)),
            pl.BlockSpec((blk, HEAD_DIM), lambda i: (i, 0)),
            pl.BlockSpec((blk, HEAD_DIM), lambda i: (i, 0)),
        ],
        out_shape=[
            jax.ShapeDtypeStruct((ORIG_LEN, ORIG_LEN), jnp.float32),
            jax.ShapeDtypeStruct((ORIG_LEN, HEAD_DIM), jnp.float32),
            jax.ShapeDtypeStruct((ORIG_LEN, HEAD_DIM), jnp.float32),
        ],
    )(cosr, sinr, msf)


def kernel(hidden_states, pos_cos, pos_sin, attention_mask, keep_ids, merged_ids):
    hs2 = hidden_states.reshape(KEPT_LEN, D_MODEL)
    cos2 = pos_cos.reshape(KEPT_LEN, HEAD_DIM)
    sin2 = pos_sin.reshape(KEPT_LEN, HEAD_DIM)
    keep2 = keep_ids.astype(jnp.int32).reshape(1, KEPT_LEN)
    r_out = jnp.arange(ORIG_LEN, dtype=jnp.int32)
    rid = jnp.concatenate([
        r_out[:IMG_START],
        IMG_START + merged_ids.astype(jnp.int32),
        r_out[IMG_START + IMG_LEN:],
    ])
    ridall = jnp.concatenate([rid, r_out]).reshape(2 * ORIG_LEN, 1)
    g2, m2 = _idx_call(keep2, ridall)
    g = g2[:ORIG_LEN, 0]
    s = g2[ORIG_LEN:, 0]
    mimg = m2[:ORIG_LEN].astype(jnp.float32)
    msrc = m2[ORIG_LEN:].astype(jnp.float32)
    out1, cosr, sinr = _sc_gather(g, s, hs2, cos2, sin2)
    hid = _stencil_call(out1, mimg)
    attn, cosf, sinf = _post_call(cosr, sinr, msrc)
    return (hid.reshape(1, ORIG_LEN, D_MODEL),
            cosf.reshape(1, ORIG_LEN, HEAD_DIM),
            sinf.reshape(1, ORIG_LEN, HEAD_DIM),
            attn.reshape(1, 1, ORIG_LEN, ORIG_LEN))


# speedup vs baseline: 11.6447x; 1.2546x over previous
import functools

import jax
import jax.numpy as jnp
from jax import lax
from jax.experimental import pallas as pl
from jax.experimental.pallas import tpu as pltpu
from jax.experimental.pallas import tpu_sc as plsc

ORIG_LEN = 4096
KEPT_LEN = 3584
IMG_START = 64
IMG_LEN = 3072
D_MODEL = 4096
HEAD_DIM = 128
H_STRIDE = 16
FRAME_STRIDE = 384

STEN_BLK = 256
HALO = 16

NC = 2
NS = 16
NW = NC * NS
ROWS_PER_W = ORIG_LEN // NW
CHUNK = 16
NCHUNK = ROWS_PER_W // CHUNK
L = 16


def _lower_bound(keep_v, x):
    pos = jnp.zeros((L,), jnp.int32)
    for bit in (2048, 1024, 512, 256, 128, 64, 32, 16, 8, 4, 2, 1):
        cand = pos + bit
        idxg = jnp.minimum(cand - 1, KEPT_LEN - 1)
        val = plsc.load_gather(keep_v, [idxg])
        ok = (cand <= KEPT_LEN) & (val < x)
        pos = jnp.where(ok, cand, pos)
    chk = plsc.load_gather(keep_v, [jnp.minimum(pos, KEPT_LEN - 1)])
    member = (pos < KEPT_LEN) & (chk == x)
    return jnp.minimum(pos, KEPT_LEN - 1), member


def _make_sc_gather():
    mesh = plsc.VectorSubcoreMesh(core_axis_name="c", subcore_axis_name="s")

    @functools.partial(
        pl.kernel,
        mesh=mesh,
        compiler_params=pltpu.CompilerParams(needs_layout_passes=False),
        out_type=[
            jax.ShapeDtypeStruct((ORIG_LEN, D_MODEL), jnp.float32),
            jax.ShapeDtypeStruct((ORIG_LEN, HEAD_DIM), jnp.float32),
            jax.ShapeDtypeStruct((ORIG_LEN, HEAD_DIM), jnp.float32),
            jax.ShapeDtypeStruct((ORIG_LEN,), jnp.float32),
            jax.ShapeDtypeStruct((ORIG_LEN,), jnp.float32),
        ],
        scratch_types=[
            pltpu.VMEM((KEPT_LEN,), jnp.int32),
            pltpu.VMEM((ROWS_PER_W,), jnp.int32),
            pltpu.VMEM((ROWS_PER_W,), jnp.int32),
            pltpu.VMEM((ROWS_PER_W,), jnp.int32),
            pltpu.VMEM((CHUNK, D_MODEL), jnp.float32),
            pltpu.VMEM((ROWS_PER_W, HEAD_DIM), jnp.float32),
            pltpu.VMEM((ROWS_PER_W,), jnp.float32),
            pltpu.VMEM((ROWS_PER_W,), jnp.float32),
            pltpu.SemaphoreType.DMA,
        ],
    )
    def sc_gather(rid_hbm, keep_hbm, hs_hbm, cos_hbm, sin_hbm,
                  out_hbm, cosr_hbm, sinr_hbm, mimg_hbm, msrc_hbm,
                  keep_v, rid_v, idxg_v, idx2_v, rows_v, crow_v,
                  mimg_v, msrc_v, sem):
        wid = lax.axis_index("s") * NC + lax.axis_index("c")
        base = wid * ROWS_PER_W
        pltpu.sync_copy(keep_hbm, keep_v)
        pltpu.sync_copy(rid_hbm.at[pl.ds(base, ROWS_PER_W)], rid_v)
        one = jnp.ones((L,), jnp.float32)
        zero = jnp.zeros((L,), jnp.float32)
        for grp in range(NCHUNK):
            rid_vec = rid_v[pl.ds(grp * L, L)]
            g_vec, m_vec = _lower_bound(keep_v, rid_vec)
            idxg_v[pl.ds(grp * L, L)] = g_vec
            mimg_v[pl.ds(grp * L, L)] = jnp.where(m_vec, one, zero)
            r_vec = base + grp * L + lax.iota(jnp.int32, L)
            s_vec, ms_vec = _lower_bound(keep_v, r_vec)
            idx2_v[pl.ds(grp * L, L)] = s_vec
            msrc_v[pl.ds(grp * L, L)] = jnp.where(ms_vec, one, zero)
        for grp in range(NCHUNK):
            pltpu.async_copy(hs_hbm.at[idxg_v.at[pl.ds(grp * CHUNK, CHUNK)]],
                             rows_v, sem).wait()
            pltpu.sync_copy(rows_v, out_hbm.at[pl.ds(base + grp * CHUNK, CHUNK)])
        pltpu.sync_copy(mimg_v, mimg_hbm.at[pl.ds(base, ROWS_PER_W)])
        pltpu.sync_copy(msrc_v, msrc_hbm.at[pl.ds(base, ROWS_PER_W)])
        pltpu.async_copy(cos_hbm.at[idx2_v], crow_v, sem).wait()
        pltpu.sync_copy(crow_v, cosr_hbm.at[pl.ds(base, ROWS_PER_W)])
        pltpu.async_copy(sin_hbm.at[idx2_v], crow_v, sem).wait()
        pltpu.sync_copy(crow_v, sinr_hbm.at[pl.ds(base, ROWS_PER_W)])

    return sc_gather


_sc_gather = _make_sc_gather()


def _stencil_body(pre_ref, main_ref, post_ref, mpre_ref, mmain_ref, mpost_ref,
                  cr_ref, sr_ref, ms_ref, out_ref, oc_ref, os_ref):
    pid = pl.program_id(0)
    Mw = jnp.concatenate([mpre_ref[...], mmain_ref[...], mpost_ref[...]],
                         axis=0)
    W = jnp.concatenate([pre_ref[...], main_ref[...], post_ref[...]], axis=0)
    W = W * Mw
    S = jnp.sign(W)
    T = jnp.concatenate(
        [jnp.zeros((HALO, 1), jnp.float32),
         jnp.sum(S[HALO:] * S[:-HALO], axis=1, keepdims=True)], axis=0)
    Lsim = jnp.concatenate(
        [jnp.zeros((1, 1), jnp.float32),
         jnp.sum(S[1:] * S[:-1], axis=1, keepdims=True)], axis=0)
    w = jax.lax.broadcasted_iota(jnp.int32, (STEN_BLK + 2 * HALO, 1), 0)
    r = pid * STEN_BLK - HALO + w
    j = r - IMG_START
    notmargin = ((j >= FRAME_STRIDE) & (j % FRAME_STRIDE >= H_STRIDE)
                 & (j % H_STRIDE != 0) & (j < IMG_LEN))
    fany = notmargin & (jnp.maximum(T, Lsim) >= 0.0)
    left_wins = Lsim > T
    f1 = (fany & left_wins).astype(jnp.float32)
    f16 = (fany & jnp.logical_not(left_wins)).astype(jnp.float32)
    Wm = W[HALO:HALO + STEN_BLK]
    f1p = f1[HALO + 1:HALO + 1 + STEN_BLK]
    f16p = f16[2 * HALO:2 * HALO + STEN_BLK]
    num = (Wm + f1p * W[HALO + 1:HALO + 1 + STEN_BLK]
           + f16p * W[2 * HALO:2 * HALO + STEN_BLK])
    den = 1.0 + f1p + f16p
    rm = r[HALO:HALO + STEN_BLK]
    is_img = (rm >= IMG_START) & (rm < IMG_START + IMG_LEN)
    fm = fany[HALO:HALO + STEN_BLK]
    merged = jnp.where(fm, 0.0, num / den)
    out_ref[...] = jnp.where(is_img, merged, Wm)
    msf = ms_ref[...]
    oc_ref[...] = cr_ref[...] * msf
    os_ref[...] = sr_ref[...] * msf


def _stencil_call(out1, mimg, cosr, sinr, msrc):
    nblk = ORIG_LEN // STEN_BLK
    nhalo = ORIG_LEN // HALO
    hpre = lambda i: (jnp.maximum(i * (STEN_BLK // HALO) - 1, 0), 0)
    hpost = lambda i: (jnp.minimum(i * (STEN_BLK // HALO) + (STEN_BLK // HALO),
                                   nhalo - 1), 0)
    main = lambda i: (i, 0)
    return pl.pallas_call(
        _stencil_body,
        grid=(nblk,),
        in_specs=[
            pl.BlockSpec((HALO, D_MODEL), hpre),
            pl.BlockSpec((STEN_BLK, D_MODEL), main),
            pl.BlockSpec((HALO, D_MODEL), hpost),
            pl.BlockSpec((HALO, 1), hpre),
            pl.BlockSpec((STEN_BLK, 1), main),
            pl.BlockSpec((HALO, 1), hpost),
            pl.BlockSpec((STEN_BLK, HEAD_DIM), main),
            pl.BlockSpec((STEN_BLK, HEAD_DIM), main),
            pl.BlockSpec((STEN_BLK, 1), main),
        ],
        out_specs=[
            pl.BlockSpec((STEN_BLK, D_MODEL), main),
            pl.BlockSpec((STEN_BLK, HEAD_DIM), main),
            pl.BlockSpec((STEN_BLK, HEAD_DIM), main),
        ],
        out_shape=[
            jax.ShapeDtypeStruct((ORIG_LEN, D_MODEL), jnp.float32),
            jax.ShapeDtypeStruct((ORIG_LEN, HEAD_DIM), jnp.float32),
            jax.ShapeDtypeStruct((ORIG_LEN, HEAD_DIM), jnp.float32),
        ],
    )(out1, out1, out1, mimg, mimg, mimg, cosr, sinr, msrc)


def _zeros_body(o_ref):
    o_ref[...] = jnp.zeros_like(o_ref)


def _zeros_call():
    return pl.pallas_call(
        _zeros_body,
        grid=(8,),
        out_specs=pl.BlockSpec((ORIG_LEN // 8, ORIG_LEN), lambda i: (i, 0)),
        out_shape=jax.ShapeDtypeStruct((ORIG_LEN, ORIG_LEN), jnp.float32),
    )()


def kernel(hidden_states, pos_cos, pos_sin, attention_mask, keep_ids, merged_ids):
    hs2 = hidden_states.reshape(KEPT_LEN, D_MODEL)
    cos2 = pos_cos.reshape(KEPT_LEN, HEAD_DIM)
    sin2 = pos_sin.reshape(KEPT_LEN, HEAD_DIM)
    keep1 = keep_ids.astype(jnp.int32)
    r_out = jnp.arange(ORIG_LEN, dtype=jnp.int32)
    rid = jnp.concatenate([
        r_out[:IMG_START],
        IMG_START + merged_ids.astype(jnp.int32),
        r_out[IMG_START + IMG_LEN:],
    ])
    attn = _zeros_call()
    out1, cosr, sinr, mimg, msrc = _sc_gather(rid, keep1, hs2, cos2, sin2)
    hid, cosf, sinf = _stencil_call(out1, mimg.reshape(ORIG_LEN, 1),
                                    cosr, sinr, msrc.reshape(ORIG_LEN, 1))
    return (hid.reshape(1, ORIG_LEN, D_MODEL),
            cosf.reshape(1, ORIG_LEN, HEAD_DIM),
            sinf.reshape(1, ORIG_LEN, HEAD_DIM),
            attn.reshape(1, 1, ORIG_LEN, ORIG_LEN))


# speedup vs baseline: 11.8905x; 1.0211x over previous
import functools

import jax
import jax.numpy as jnp
from jax import lax
from jax.experimental import pallas as pl
from jax.experimental.pallas import tpu as pltpu
from jax.experimental.pallas import tpu_sc as plsc

ORIG_LEN = 4096
KEPT_LEN = 3584
IMG_START = 64
IMG_LEN = 3072
D_MODEL = 4096
HEAD_DIM = 128
H_STRIDE = 16
FRAME_STRIDE = 384

STEN_BLK = 256
HALO = 16

NC = 2
NS = 16
NW = NC * NS
ROWS_PER_W = ORIG_LEN // NW
CHUNK = 8
NCHUNK = ROWS_PER_W // CHUNK
NBUF = 2
NGRP = ROWS_PER_W // 16
L = 16


def _lower_bound(keep_v, x):
    pos = jnp.zeros((L,), jnp.int32)
    for bit in (2048, 1024, 512, 256, 128, 64, 32, 16, 8, 4, 2, 1):
        cand = pos + bit
        idxg = jnp.minimum(cand - 1, KEPT_LEN - 1)
        val = plsc.load_gather(keep_v, [idxg])
        ok = (cand <= KEPT_LEN) & (val < x)
        pos = jnp.where(ok, cand, pos)
    chk = plsc.load_gather(keep_v, [jnp.minimum(pos, KEPT_LEN - 1)])
    member = (pos < KEPT_LEN) & (chk == x)
    return jnp.minimum(pos, KEPT_LEN - 1), member


def _make_sc_gather():
    mesh = plsc.VectorSubcoreMesh(core_axis_name="c", subcore_axis_name="s")

    @functools.partial(
        pl.kernel,
        mesh=mesh,
        compiler_params=pltpu.CompilerParams(needs_layout_passes=False),
        out_type=[
            jax.ShapeDtypeStruct((ORIG_LEN, D_MODEL), jnp.float32),
            jax.ShapeDtypeStruct((ORIG_LEN, HEAD_DIM), jnp.float32),
            jax.ShapeDtypeStruct((ORIG_LEN, HEAD_DIM), jnp.float32),
            jax.ShapeDtypeStruct((ORIG_LEN,), jnp.float32),
            jax.ShapeDtypeStruct((ORIG_LEN,), jnp.float32),
        ],
        scratch_types=[
            pltpu.VMEM((KEPT_LEN,), jnp.int32),
            pltpu.VMEM((ROWS_PER_W,), jnp.int32),
            pltpu.VMEM((ROWS_PER_W,), jnp.int32),
            pltpu.VMEM((ROWS_PER_W,), jnp.int32),
            pltpu.VMEM((NBUF * CHUNK, D_MODEL), jnp.float32),
            pltpu.VMEM((ROWS_PER_W, HEAD_DIM), jnp.float32),
            pltpu.VMEM((ROWS_PER_W,), jnp.float32),
            pltpu.VMEM((ROWS_PER_W,), jnp.float32),
            pltpu.SemaphoreType.DMA,
            pltpu.SemaphoreType.DMA,
            pltpu.SemaphoreType.DMA,
            pltpu.SemaphoreType.DMA,
            pltpu.SemaphoreType.DMA,
        ],
    )
    def sc_gather(rid_hbm, keep_hbm, hs_hbm, cos_hbm, sin_hbm,
                  out_hbm, cosr_hbm, sinr_hbm, mimg_hbm, msrc_hbm,
                  keep_v, rid_v, idxg_v, idx2_v, rows_v, crow_v,
                  mimg_v, msrc_v, gsem0, gsem1, ssem0, ssem1, sem):
        wid = lax.axis_index("s") * NC + lax.axis_index("c")
        base = wid * ROWS_PER_W
        pltpu.sync_copy(keep_hbm, keep_v)
        pltpu.sync_copy(rid_hbm.at[pl.ds(base, ROWS_PER_W)], rid_v)
        one = jnp.ones((L,), jnp.float32)
        zero = jnp.zeros((L,), jnp.float32)
        for grp in range(NGRP):
            rid_vec = rid_v[pl.ds(grp * L, L)]
            g_vec, m_vec = _lower_bound(keep_v, rid_vec)
            idxg_v[pl.ds(grp * L, L)] = g_vec
            mimg_v[pl.ds(grp * L, L)] = jnp.where(m_vec, one, zero)
            r_vec = base + grp * L + lax.iota(jnp.int32, L)
            s_vec, ms_vec = _lower_bound(keep_v, r_vec)
            idx2_v[pl.ds(grp * L, L)] = s_vec
            msrc_v[pl.ds(grp * L, L)] = jnp.where(ms_vec, one, zero)
        gsems = (gsem0, gsem1)
        ssems = (ssem0, ssem1)

        def _gstart(g, b):
            return pltpu.async_copy(
                hs_hbm.at[idxg_v.at[pl.ds(g * CHUNK, CHUNK)]],
                rows_v.at[pl.ds(b * CHUNK, CHUNK)], gsems[b])

        def _sstart(g, b):
            return pltpu.async_copy(
                rows_v.at[pl.ds(b * CHUNK, CHUNK)],
                out_hbm.at[pl.ds(base + g * CHUNK, CHUNK)], ssems[b])

        copies = [None] * NCHUNK
        stores = [None] * NCHUNK
        copies[0] = _gstart(0, 0)
        for g in range(NCHUNK):
            b = g % NBUF
            if g + 1 < NCHUNK:
                if g + 1 >= NBUF:
                    stores[g + 1 - NBUF].wait()
                copies[g + 1] = _gstart(g + 1, (g + 1) % NBUF)
            copies[g].wait()
            stores[g] = _sstart(g, b)
        for g in range(max(0, NCHUNK - NBUF), NCHUNK):
            stores[g].wait()
        pltpu.sync_copy(mimg_v, mimg_hbm.at[pl.ds(base, ROWS_PER_W)])
        pltpu.sync_copy(msrc_v, msrc_hbm.at[pl.ds(base, ROWS_PER_W)])
        pltpu.async_copy(cos_hbm.at[idx2_v], crow_v, sem).wait()
        pltpu.sync_copy(crow_v, cosr_hbm.at[pl.ds(base, ROWS_PER_W)])
        pltpu.async_copy(sin_hbm.at[idx2_v], crow_v, sem).wait()
        pltpu.sync_copy(crow_v, sinr_hbm.at[pl.ds(base, ROWS_PER_W)])

    return sc_gather


_sc_gather = _make_sc_gather()


def _stencil_body(pre_ref, main_ref, post_ref, mpre_ref, mmain_ref, mpost_ref,
                  cr_ref, sr_ref, ms_ref, out_ref, oc_ref, os_ref):
    pid = pl.program_id(0)
    Mw = jnp.concatenate([mpre_ref[...], mmain_ref[...], mpost_ref[...]],
                         axis=0)
    W = jnp.concatenate([pre_ref[...], main_ref[...], post_ref[...]], axis=0)
    W = W * Mw
    S = jnp.sign(W)
    T = jnp.concatenate(
        [jnp.zeros((HALO, 1), jnp.float32),
         jnp.sum(S[HALO:] * S[:-HALO], axis=1, keepdims=True)], axis=0)
    Lsim = jnp.concatenate(
        [jnp.zeros((1, 1), jnp.float32),
         jnp.sum(S[1:] * S[:-1], axis=1, keepdims=True)], axis=0)
    w = jax.lax.broadcasted_iota(jnp.int32, (STEN_BLK + 2 * HALO, 1), 0)
    r = pid * STEN_BLK - HALO + w
    j = r - IMG_START
    notmargin = ((j >= FRAME_STRIDE) & (j % FRAME_STRIDE >= H_STRIDE)
                 & (j % H_STRIDE != 0) & (j < IMG_LEN))
    fany = notmargin & (jnp.maximum(T, Lsim) >= 0.0)
    left_wins = Lsim > T
    f1 = (fany & left_wins).astype(jnp.float32)
    f16 = (fany & jnp.logical_not(left_wins)).astype(jnp.float32)
    Wm = W[HALO:HALO + STEN_BLK]
    f1p = f1[HALO + 1:HALO + 1 + STEN_BLK]
    f16p = f16[2 * HALO:2 * HALO + STEN_BLK]
    num = (Wm + f1p * W[HALO + 1:HALO + 1 + STEN_BLK]
           + f16p * W[2 * HALO:2 * HALO + STEN_BLK])
    den = 1.0 + f1p + f16p
    rm = r[HALO:HALO + STEN_BLK]
    is_img = (rm >= IMG_START) & (rm < IMG_START + IMG_LEN)
    fm = fany[HALO:HALO + STEN_BLK]
    merged = jnp.where(fm, 0.0, num / den)
    out_ref[...] = jnp.where(is_img, merged, Wm)
    msf = ms_ref[...]
    oc_ref[...] = cr_ref[...] * msf
    os_ref[...] = sr_ref[...] * msf


def _stencil_call(out1, mimg, cosr, sinr, msrc):
    nblk = ORIG_LEN // STEN_BLK
    nhalo = ORIG_LEN // HALO
    hpre = lambda i: (jnp.maximum(i * (STEN_BLK // HALO) - 1, 0), 0)
    hpost = lambda i: (jnp.minimum(i * (STEN_BLK // HALO) + (STEN_BLK // HALO),
                                   nhalo - 1), 0)
    main = lambda i: (i, 0)
    return pl.pallas_call(
        _stencil_body,
        grid=(nblk,),
        in_specs=[
            pl.BlockSpec((HALO, D_MODEL), hpre),
            pl.BlockSpec((STEN_BLK, D_MODEL), main),
            pl.BlockSpec((HALO, D_MODEL), hpost),
            pl.BlockSpec((HALO, 1), hpre),
            pl.BlockSpec((STEN_BLK, 1), main),
            pl.BlockSpec((HALO, 1), hpost),
            pl.BlockSpec((STEN_BLK, HEAD_DIM), main),
            pl.BlockSpec((STEN_BLK, HEAD_DIM), main),
            pl.BlockSpec((STEN_BLK, 1), main),
        ],
        out_specs=[
            pl.BlockSpec((STEN_BLK, D_MODEL), main),
            pl.BlockSpec((STEN_BLK, HEAD_DIM), main),
            pl.BlockSpec((STEN_BLK, HEAD_DIM), main),
        ],
        out_shape=[
            jax.ShapeDtypeStruct((ORIG_LEN, D_MODEL), jnp.float32),
            jax.ShapeDtypeStruct((ORIG_LEN, HEAD_DIM), jnp.float32),
            jax.ShapeDtypeStruct((ORIG_LEN, HEAD_DIM), jnp.float32),
        ],
    )(out1, out1, out1, mimg, mimg, mimg, cosr, sinr, msrc)


def _zeros_body(o_ref):
    o_ref[...] = jnp.zeros_like(o_ref)


def _zeros_call():
    return pl.pallas_call(
        _zeros_body,
        grid=(8,),
        out_specs=pl.BlockSpec((ORIG_LEN // 8, ORIG_LEN), lambda i: (i, 0)),
        out_shape=jax.ShapeDtypeStruct((ORIG_LEN, ORIG_LEN), jnp.float32),
    )()


def kernel(hidden_states, pos_cos, pos_sin, attention_mask, keep_ids, merged_ids):
    hs2 = hidden_states.reshape(KEPT_LEN, D_MODEL)
    cos2 = pos_cos.reshape(KEPT_LEN, HEAD_DIM)
    sin2 = pos_sin.reshape(KEPT_LEN, HEAD_DIM)
    keep1 = keep_ids.astype(jnp.int32)
    r_out = jnp.arange(ORIG_LEN, dtype=jnp.int32)
    rid = jnp.concatenate([
        r_out[:IMG_START],
        IMG_START + merged_ids.astype(jnp.int32),
        r_out[IMG_START + IMG_LEN:],
    ])
    attn = _zeros_call()
    out1, cosr, sinr, mimg, msrc = _sc_gather(rid, keep1, hs2, cos2, sin2)
    hid, cosf, sinf = _stencil_call(out1, mimg.reshape(ORIG_LEN, 1),
                                    cosr, sinr, msrc.reshape(ORIG_LEN, 1))
    return (hid.reshape(1, ORIG_LEN, D_MODEL),
            cosf.reshape(1, ORIG_LEN, HEAD_DIM),
            sinf.reshape(1, ORIG_LEN, HEAD_DIM),
            attn.reshape(1, 1, ORIG_LEN, ORIG_LEN))


# speedup vs baseline: 13.3016x; 1.1187x over previous
import functools

import jax
import jax.numpy as jnp
from jax import lax
from jax.experimental import pallas as pl
from jax.experimental.pallas import tpu as pltpu
from jax.experimental.pallas import tpu_sc as plsc

ORIG_LEN = 4096
KEPT_LEN = 3584
IMG_START = 64
IMG_LEN = 3072
D_MODEL = 4096
HEAD_DIM = 128
H_STRIDE = 16
FRAME_STRIDE = 384

STEN_BLK = 256
HALO = 16

NC = 2
NS = 16
NW = NC * NS
ROWS_PER_W = ORIG_LEN // NW
CHUNK = 8
NCHUNK = ROWS_PER_W // CHUNK
NBUF = 2
NGRP = ROWS_PER_W // 16
L = 16


def _lower_bound(keep_v, x):
    pos = jnp.zeros((L,), jnp.int32)
    for bit in (2048, 1024, 512, 256, 128, 64, 32, 16, 8, 4, 2, 1):
        cand = pos + bit
        idxg = jnp.minimum(cand - 1, KEPT_LEN - 1)
        val = plsc.load_gather(keep_v, [idxg])
        ok = (cand <= KEPT_LEN) & (val < x)
        pos = jnp.where(ok, cand, pos)
    chk = plsc.load_gather(keep_v, [jnp.minimum(pos, KEPT_LEN - 1)])
    member = (pos < KEPT_LEN) & (chk == x)
    return jnp.minimum(pos, KEPT_LEN - 1), member


def _make_sc_gather():
    mesh = plsc.VectorSubcoreMesh(core_axis_name="c", subcore_axis_name="s")

    @functools.partial(
        pl.kernel,
        mesh=mesh,
        compiler_params=pltpu.CompilerParams(needs_layout_passes=False),
        out_type=[
            jax.ShapeDtypeStruct((ORIG_LEN, D_MODEL), jnp.float32),
            jax.ShapeDtypeStruct((ORIG_LEN, HEAD_DIM), jnp.float32),
            jax.ShapeDtypeStruct((ORIG_LEN, HEAD_DIM), jnp.float32),
            jax.ShapeDtypeStruct((ORIG_LEN,), jnp.float32),
            jax.ShapeDtypeStruct((ORIG_LEN,), jnp.float32),
        ],
        scratch_types=[
            pltpu.VMEM((KEPT_LEN,), jnp.int32),
            pltpu.VMEM((ROWS_PER_W,), jnp.int32),
            pltpu.VMEM((ROWS_PER_W,), jnp.int32),
            pltpu.VMEM((ROWS_PER_W,), jnp.int32),
            pltpu.VMEM((NBUF * CHUNK, D_MODEL), jnp.float32),
            pltpu.VMEM((ROWS_PER_W, HEAD_DIM), jnp.float32),
            pltpu.VMEM((ROWS_PER_W,), jnp.float32),
            pltpu.VMEM((ROWS_PER_W,), jnp.float32),
            pltpu.SemaphoreType.DMA,
            pltpu.SemaphoreType.DMA,
            pltpu.SemaphoreType.DMA,
            pltpu.SemaphoreType.DMA,
            pltpu.SemaphoreType.DMA,
        ],
    )
    def sc_gather(rid_hbm, keep_hbm, hs_hbm, cos_hbm, sin_hbm,
                  out_hbm, cosr_hbm, sinr_hbm, mimg_hbm, msrc_hbm,
                  keep_v, rid_v, idxg_v, idx2_v, rows_v, crow_v,
                  mimg_v, msrc_v, gsem0, gsem1, ssem0, ssem1, sem):
        wid = lax.axis_index("s") * NC + lax.axis_index("c")
        base = wid * ROWS_PER_W
        pltpu.sync_copy(keep_hbm, keep_v)
        pltpu.sync_copy(rid_hbm.at[pl.ds(base, ROWS_PER_W)], rid_v)
        one = jnp.ones((L,), jnp.float32)
        zero = jnp.zeros((L,), jnp.float32)
        for grp in range(NGRP):
            rid_vec = rid_v[pl.ds(grp * L, L)]
            g_vec, m_vec = _lower_bound(keep_v, rid_vec)
            idxg_v[pl.ds(grp * L, L)] = g_vec
            mimg_v[pl.ds(grp * L, L)] = jnp.where(m_vec, one, zero)
            r_vec = base + grp * L + lax.iota(jnp.int32, L)
            s_vec, ms_vec = _lower_bound(keep_v, r_vec)
            idx2_v[pl.ds(grp * L, L)] = s_vec
            msrc_v[pl.ds(grp * L, L)] = jnp.where(ms_vec, one, zero)
        gsems = (gsem0, gsem1)
        ssems = (ssem0, ssem1)

        def _gstart(g, b):
            return pltpu.async_copy(
                hs_hbm.at[idxg_v.at[pl.ds(g * CHUNK, CHUNK)]],
                rows_v.at[pl.ds(b * CHUNK, CHUNK)], gsems[b])

        def _sstart(g, b):
            return pltpu.async_copy(
                rows_v.at[pl.ds(b * CHUNK, CHUNK)],
                out_hbm.at[pl.ds(base + g * CHUNK, CHUNK)], ssems[b])

        copies = [None] * NCHUNK
        stores = [None] * NCHUNK
        copies[0] = _gstart(0, 0)
        for g in range(NCHUNK):
            b = g % NBUF
            if g + 1 < NCHUNK:
                if g + 1 >= NBUF:
                    stores[g + 1 - NBUF].wait()
                copies[g + 1] = _gstart(g + 1, (g + 1) % NBUF)
            copies[g].wait()
            stores[g] = _sstart(g, b)
        for g in range(max(0, NCHUNK - NBUF), NCHUNK):
            stores[g].wait()
        pltpu.sync_copy(mimg_v, mimg_hbm.at[pl.ds(base, ROWS_PER_W)])
        pltpu.sync_copy(msrc_v, msrc_hbm.at[pl.ds(base, ROWS_PER_W)])
        pltpu.async_copy(cos_hbm.at[idx2_v], crow_v, sem).wait()
        pltpu.sync_copy(crow_v, cosr_hbm.at[pl.ds(base, ROWS_PER_W)])
        pltpu.async_copy(sin_hbm.at[idx2_v], crow_v, sem).wait()
        pltpu.sync_copy(crow_v, sinr_hbm.at[pl.ds(base, ROWS_PER_W)])

    return sc_gather


_sc_gather = _make_sc_gather()


def _stencil_body(pre_ref, main_ref, post_ref, mpre_ref, mmain_ref, mpost_ref,
                  cr_ref, sr_ref, ms_ref, out_ref, oc_ref, os_ref):
    pid = pl.program_id(0)
    Mw = jnp.concatenate([mpre_ref[...], mmain_ref[...], mpost_ref[...]],
                         axis=0)
    W = jnp.concatenate([pre_ref[...], main_ref[...], post_ref[...]], axis=0)
    W = W * Mw
    S = jnp.sign(W).astype(jnp.bfloat16)
    M = jax.lax.dot_general(S, S, (((1,), (1,)), ((), ())),
                            preferred_element_type=jnp.float32)
    WIN = STEN_BLK + 2 * HALO
    row = jax.lax.broadcasted_iota(jnp.int32, (WIN, WIN), 0)
    col = jax.lax.broadcasted_iota(jnp.int32, (WIN, WIN), 1)
    T = jnp.sum(jnp.where(col == row - HALO, M, 0.0), axis=1, keepdims=True)
    Lsim = jnp.sum(jnp.where(col == row - 1, M, 0.0), axis=1, keepdims=True)
    w = jax.lax.broadcasted_iota(jnp.int32, (STEN_BLK + 2 * HALO, 1), 0)
    r = pid * STEN_BLK - HALO + w
    j = r - IMG_START
    notmargin = ((j >= FRAME_STRIDE) & (j % FRAME_STRIDE >= H_STRIDE)
                 & (j % H_STRIDE != 0) & (j < IMG_LEN))
    fany = notmargin & (jnp.maximum(T, Lsim) >= 0.0)
    left_wins = Lsim > T
    f1 = (fany & left_wins).astype(jnp.float32)
    f16 = (fany & jnp.logical_not(left_wins)).astype(jnp.float32)
    Wm = W[HALO:HALO + STEN_BLK]
    f1p = f1[HALO + 1:HALO + 1 + STEN_BLK]
    f16p = f16[2 * HALO:2 * HALO + STEN_BLK]
    num = (Wm + f1p * W[HALO + 1:HALO + 1 + STEN_BLK]
           + f16p * W[2 * HALO:2 * HALO + STEN_BLK])
    den = 1.0 + f1p + f16p
    rm = r[HALO:HALO + STEN_BLK]
    is_img = (rm >= IMG_START) & (rm < IMG_START + IMG_LEN)
    fm = fany[HALO:HALO + STEN_BLK]
    merged = jnp.where(fm, 0.0, num / den)
    out_ref[...] = jnp.where(is_img, merged, Wm)
    msf = ms_ref[...]
    oc_ref[...] = cr_ref[...] * msf
    os_ref[...] = sr_ref[...] * msf


def _stencil_call(out1, mimg, cosr, sinr, msrc):
    nblk = ORIG_LEN // STEN_BLK
    nhalo = ORIG_LEN // HALO
    hpre = lambda i: (jnp.maximum(i * (STEN_BLK // HALO) - 1, 0), 0)
    hpost = lambda i: (jnp.minimum(i * (STEN_BLK // HALO) + (STEN_BLK // HALO),
                                   nhalo - 1), 0)
    main = lambda i: (i, 0)
    return pl.pallas_call(
        _stencil_body,
        grid=(nblk,),
        in_specs=[
            pl.BlockSpec((HALO, D_MODEL), hpre),
            pl.BlockSpec((STEN_BLK, D_MODEL), main),
            pl.BlockSpec((HALO, D_MODEL), hpost),
            pl.BlockSpec((HALO, 1), hpre),
            pl.BlockSpec((STEN_BLK, 1), main),
            pl.BlockSpec((HALO, 1), hpost),
            pl.BlockSpec((STEN_BLK, HEAD_DIM), main),
            pl.BlockSpec((STEN_BLK, HEAD_DIM), main),
            pl.BlockSpec((STEN_BLK, 1), main),
        ],
        out_specs=[
            pl.BlockSpec((STEN_BLK, D_MODEL), main),
            pl.BlockSpec((STEN_BLK, HEAD_DIM), main),
            pl.BlockSpec((STEN_BLK, HEAD_DIM), main),
        ],
        out_shape=[
            jax.ShapeDtypeStruct((ORIG_LEN, D_MODEL), jnp.float32),
            jax.ShapeDtypeStruct((ORIG_LEN, HEAD_DIM), jnp.float32),
            jax.ShapeDtypeStruct((ORIG_LEN, HEAD_DIM), jnp.float32),
        ],
    )(out1, out1, out1, mimg, mimg, mimg, cosr, sinr, msrc)


def _zeros_body(o_ref):
    o_ref[...] = jnp.zeros_like(o_ref)


def _zeros_call():
    return pl.pallas_call(
        _zeros_body,
        grid=(8,),
        out_specs=pl.BlockSpec((ORIG_LEN // 8, ORIG_LEN), lambda i: (i, 0)),
        out_shape=jax.ShapeDtypeStruct((ORIG_LEN, ORIG_LEN), jnp.float32),
    )()


def kernel(hidden_states, pos_cos, pos_sin, attention_mask, keep_ids, merged_ids):
    hs2 = hidden_states.reshape(KEPT_LEN, D_MODEL)
    cos2 = pos_cos.reshape(KEPT_LEN, HEAD_DIM)
    sin2 = pos_sin.reshape(KEPT_LEN, HEAD_DIM)
    keep1 = keep_ids.astype(jnp.int32)
    r_out = jnp.arange(ORIG_LEN, dtype=jnp.int32)
    rid = jnp.concatenate([
        r_out[:IMG_START],
        IMG_START + merged_ids.astype(jnp.int32),
        r_out[IMG_START + IMG_LEN:],
    ])
    attn = _zeros_call()
    out1, cosr, sinr, mimg, msrc = _sc_gather(rid, keep1, hs2, cos2, sin2)
    hid, cosf, sinf = _stencil_call(out1, mimg.reshape(ORIG_LEN, 1),
                                    cosr, sinr, msrc.reshape(ORIG_LEN, 1))
    return (hid.reshape(1, ORIG_LEN, D_MODEL),
            cosf.reshape(1, ORIG_LEN, HEAD_DIM),
            sinf.reshape(1, ORIG_LEN, HEAD_DIM),
            attn.reshape(1, 1, ORIG_LEN, ORIG_LEN))


# speedup vs baseline: 13.3718x; 1.0053x over previous
import functools

import jax
import jax.numpy as jnp
from jax import lax
from jax.experimental import pallas as pl
from jax.experimental.pallas import tpu as pltpu
from jax.experimental.pallas import tpu_sc as plsc

ORIG_LEN = 4096
KEPT_LEN = 3584
IMG_START = 64
IMG_LEN = 3072
D_MODEL = 4096
HEAD_DIM = 128
H_STRIDE = 16
FRAME_STRIDE = 384

STEN_BLK = 256
HALO = 16

NC = 2
NS = 16
NW = NC * NS
ROWS_PER_W = ORIG_LEN // NW
CHUNK = 8
NCHUNK = ROWS_PER_W // CHUNK
NBUF = 2
NGRP = ROWS_PER_W // 16
L = 16


def _lower_bound(keep_v, x):
    pos = jnp.zeros((L,), jnp.int32)
    for bit in (2048, 1024, 512, 256, 128, 64, 32, 16, 8, 4, 2, 1):
        cand = pos + bit
        idxg = jnp.minimum(cand - 1, KEPT_LEN - 1)
        val = plsc.load_gather(keep_v, [idxg])
        ok = (cand <= KEPT_LEN) & (val < x)
        pos = jnp.where(ok, cand, pos)
    chk = plsc.load_gather(keep_v, [jnp.minimum(pos, KEPT_LEN - 1)])
    member = (pos < KEPT_LEN) & (chk == x)
    return jnp.minimum(pos, KEPT_LEN - 1), member


def _make_sc_gather():
    mesh = plsc.VectorSubcoreMesh(core_axis_name="c", subcore_axis_name="s")

    @functools.partial(
        pl.kernel,
        mesh=mesh,
        compiler_params=pltpu.CompilerParams(needs_layout_passes=False),
        out_type=[
            jax.ShapeDtypeStruct((ORIG_LEN, D_MODEL), jnp.float32),
            jax.ShapeDtypeStruct((ORIG_LEN, HEAD_DIM), jnp.float32),
            jax.ShapeDtypeStruct((ORIG_LEN, HEAD_DIM), jnp.float32),
            jax.ShapeDtypeStruct((ORIG_LEN,), jnp.float32),
            jax.ShapeDtypeStruct((ORIG_LEN,), jnp.float32),
        ],
        scratch_types=[
            pltpu.VMEM((KEPT_LEN,), jnp.int32),
            pltpu.VMEM((ROWS_PER_W,), jnp.int32),
            pltpu.VMEM((ROWS_PER_W,), jnp.int32),
            pltpu.VMEM((ROWS_PER_W,), jnp.int32),
            pltpu.VMEM((NBUF * CHUNK, D_MODEL), jnp.float32),
            pltpu.VMEM((ROWS_PER_W, HEAD_DIM), jnp.float32),
            pltpu.VMEM((ROWS_PER_W,), jnp.float32),
            pltpu.VMEM((ROWS_PER_W,), jnp.float32),
            pltpu.SemaphoreType.DMA,
            pltpu.SemaphoreType.DMA,
            pltpu.SemaphoreType.DMA,
            pltpu.SemaphoreType.DMA,
            pltpu.SemaphoreType.DMA,
        ],
    )
    def sc_gather(rid_hbm, keep_hbm, hs_hbm, cos_hbm, sin_hbm,
                  out_hbm, cosr_hbm, sinr_hbm, mimg_hbm, msrc_hbm,
                  keep_v, rid_v, idxg_v, idx2_v, rows_v, crow_v,
                  mimg_v, msrc_v, gsem0, gsem1, ssem0, ssem1, sem):
        wid = lax.axis_index("s") * NC + lax.axis_index("c")
        base = wid * ROWS_PER_W
        pltpu.sync_copy(keep_hbm, keep_v)
        pltpu.sync_copy(rid_hbm.at[pl.ds(base, ROWS_PER_W)], rid_v)
        one = jnp.ones((L,), jnp.float32)
        zero = jnp.zeros((L,), jnp.float32)
        for grp in range(NGRP):
            rid_vec = rid_v[pl.ds(grp * L, L)]
            g_vec, m_vec = _lower_bound(keep_v, rid_vec)
            idxg_v[pl.ds(grp * L, L)] = g_vec
            mimg_v[pl.ds(grp * L, L)] = jnp.where(m_vec, one, zero)
            r_vec = base + grp * L + lax.iota(jnp.int32, L)
            s_vec, ms_vec = _lower_bound(keep_v, r_vec)
            idx2_v[pl.ds(grp * L, L)] = s_vec
            msrc_v[pl.ds(grp * L, L)] = jnp.where(ms_vec, one, zero)
        gsems = (gsem0, gsem1)
        ssems = (ssem0, ssem1)

        def _gstart(g, b):
            return pltpu.async_copy(
                hs_hbm.at[idxg_v.at[pl.ds(g * CHUNK, CHUNK)]],
                rows_v.at[pl.ds(b * CHUNK, CHUNK)], gsems[b])

        def _sstart(g, b):
            return pltpu.async_copy(
                rows_v.at[pl.ds(b * CHUNK, CHUNK)],
                out_hbm.at[pl.ds(base + g * CHUNK, CHUNK)], ssems[b])

        copies = [None] * NCHUNK
        stores = [None] * NCHUNK
        copies[0] = _gstart(0, 0)
        for g in range(NCHUNK):
            b = g % NBUF
            if g + 1 < NCHUNK:
                if g + 1 >= NBUF:
                    stores[g + 1 - NBUF].wait()
                copies[g + 1] = _gstart(g + 1, (g + 1) % NBUF)
            copies[g].wait()
            stores[g] = _sstart(g, b)
        for g in range(max(0, NCHUNK - NBUF), NCHUNK):
            stores[g].wait()
        pltpu.sync_copy(mimg_v, mimg_hbm.at[pl.ds(base, ROWS_PER_W)])
        pltpu.sync_copy(msrc_v, msrc_hbm.at[pl.ds(base, ROWS_PER_W)])
        pltpu.async_copy(cos_hbm.at[idx2_v], crow_v, sem).wait()
        pltpu.sync_copy(crow_v, cosr_hbm.at[pl.ds(base, ROWS_PER_W)])
        pltpu.async_copy(sin_hbm.at[idx2_v], crow_v, sem).wait()
        pltpu.sync_copy(crow_v, sinr_hbm.at[pl.ds(base, ROWS_PER_W)])

    return sc_gather


_sc_gather = _make_sc_gather()


def _stencil_body(pre_ref, main_ref, post_ref, mpre_ref, mmain_ref, mpost_ref,
                  cr_ref, sr_ref, ms_ref, out_ref, oc_ref, os_ref, attn_ref):
    pid = pl.program_id(0)
    Mw = jnp.concatenate([mpre_ref[...], mmain_ref[...], mpost_ref[...]],
                         axis=0)
    W = jnp.concatenate([pre_ref[...], main_ref[...], post_ref[...]], axis=0)
    W = W * Mw
    S = jnp.sign(W).astype(jnp.bfloat16)
    M = jax.lax.dot_general(S, S, (((1,), (1,)), ((), ())),
                            preferred_element_type=jnp.float32)
    WIN = STEN_BLK + 2 * HALO
    row = jax.lax.broadcasted_iota(jnp.int32, (WIN, WIN), 0)
    col = jax.lax.broadcasted_iota(jnp.int32, (WIN, WIN), 1)
    T = jnp.sum(jnp.where(col == row - HALO, M, 0.0), axis=1, keepdims=True)
    Lsim = jnp.sum(jnp.where(col == row - 1, M, 0.0), axis=1, keepdims=True)
    w = jax.lax.broadcasted_iota(jnp.int32, (STEN_BLK + 2 * HALO, 1), 0)
    r = pid * STEN_BLK - HALO + w
    j = r - IMG_START
    notmargin = ((j >= FRAME_STRIDE) & (j % FRAME_STRIDE >= H_STRIDE)
                 & (j % H_STRIDE != 0) & (j < IMG_LEN))
    fany = notmargin & (jnp.maximum(T, Lsim) >= 0.0)
    left_wins = Lsim > T
    f1 = (fany & left_wins).astype(jnp.float32)
    f16 = (fany & jnp.logical_not(left_wins)).astype(jnp.float32)
    Wm = W[HALO:HALO + STEN_BLK]
    f1p = f1[HALO + 1:HALO + 1 + STEN_BLK]
    f16p = f16[2 * HALO:2 * HALO + STEN_BLK]
    num = (Wm + f1p * W[HALO + 1:HALO + 1 + STEN_BLK]
           + f16p * W[2 * HALO:2 * HALO + STEN_BLK])
    den = 1.0 + f1p + f16p
    rm = r[HALO:HALO + STEN_BLK]
    is_img = (rm >= IMG_START) & (rm < IMG_START + IMG_LEN)
    fm = fany[HALO:HALO + STEN_BLK]
    merged = jnp.where(fm, 0.0, num / den)
    out_ref[...] = jnp.where(is_img, merged, Wm)
    msf = ms_ref[...]
    oc_ref[...] = cr_ref[...] * msf
    os_ref[...] = sr_ref[...] * msf
    attn_ref[...] = jnp.zeros_like(attn_ref)


def _stencil_call(out1, mimg, cosr, sinr, msrc):
    nblk = ORIG_LEN // STEN_BLK
    nhalo = ORIG_LEN // HALO
    hpre = lambda i: (jnp.maximum(i * (STEN_BLK // HALO) - 1, 0), 0)
    hpost = lambda i: (jnp.minimum(i * (STEN_BLK // HALO) + (STEN_BLK // HALO),
                                   nhalo - 1), 0)
    main = lambda i: (i, 0)
    return pl.pallas_call(
        _stencil_body,
        grid=(nblk,),
        in_specs=[
            pl.BlockSpec((HALO, D_MODEL), hpre),
            pl.BlockSpec((STEN_BLK, D_MODEL), main),
            pl.BlockSpec((HALO, D_MODEL), hpost),
            pl.BlockSpec((HALO, 1), hpre),
            pl.BlockSpec((STEN_BLK, 1), main),
            pl.BlockSpec((HALO, 1), hpost),
            pl.BlockSpec((STEN_BLK, HEAD_DIM), main),
            pl.BlockSpec((STEN_BLK, HEAD_DIM), main),
            pl.BlockSpec((STEN_BLK, 1), main),
        ],
        out_specs=[
            pl.BlockSpec((STEN_BLK, D_MODEL), main),
            pl.BlockSpec((STEN_BLK, HEAD_DIM), main),
            pl.BlockSpec((STEN_BLK, HEAD_DIM), main),
            pl.BlockSpec((STEN_BLK, ORIG_LEN), main),
        ],
        out_shape=[
            jax.ShapeDtypeStruct((ORIG_LEN, D_MODEL), jnp.float32),
            jax.ShapeDtypeStruct((ORIG_LEN, HEAD_DIM), jnp.float32),
            jax.ShapeDtypeStruct((ORIG_LEN, HEAD_DIM), jnp.float32),
            jax.ShapeDtypeStruct((ORIG_LEN, ORIG_LEN), jnp.float32),
        ],
    )(out1, out1, out1, mimg, mimg, mimg, cosr, sinr, msrc)


def _zeros_body(o_ref):
    o_ref[...] = jnp.zeros_like(o_ref)


def _zeros_call():
    return pl.pallas_call(
        _zeros_body,
        grid=(8,),
        out_specs=pl.BlockSpec((ORIG_LEN // 8, ORIG_LEN), lambda i: (i, 0)),
        out_shape=jax.ShapeDtypeStruct((ORIG_LEN, ORIG_LEN), jnp.float32),
    )()


def kernel(hidden_states, pos_cos, pos_sin, attention_mask, keep_ids, merged_ids):
    hs2 = hidden_states.reshape(KEPT_LEN, D_MODEL)
    cos2 = pos_cos.reshape(KEPT_LEN, HEAD_DIM)
    sin2 = pos_sin.reshape(KEPT_LEN, HEAD_DIM)
    keep1 = keep_ids.astype(jnp.int32)
    r_out = jnp.arange(ORIG_LEN, dtype=jnp.int32)
    rid = jnp.concatenate([
        r_out[:IMG_START],
        IMG_START + merged_ids.astype(jnp.int32),
        r_out[IMG_START + IMG_LEN:],
    ])
    out1, cosr, sinr, mimg, msrc = _sc_gather(rid, keep1, hs2, cos2, sin2)
    hid, cosf, sinf, attn = _stencil_call(out1, mimg.reshape(ORIG_LEN, 1),
                                          cosr, sinr, msrc.reshape(ORIG_LEN, 1))
    return (hid.reshape(1, ORIG_LEN, D_MODEL),
            cosf.reshape(1, ORIG_LEN, HEAD_DIM),
            sinf.reshape(1, ORIG_LEN, HEAD_DIM),
            attn.reshape(1, 1, ORIG_LEN, ORIG_LEN))
